# baseline (speedup 1.0000x reference)
import functools

import jax
import jax.numpy as jnp
from jax import lax
from jax.experimental import pallas as pl
from jax.experimental.pallas import tpu as pltpu
from jax.experimental.pallas import tpu_sc as plsc

_NC = 2
_NS = 16
_NW = _NC * _NS
_L = 16


@functools.lru_cache(maxsize=None)
def _make_gather(N, D, B):
    b_per_w = B // _NW
    C = 256
    n_chunks = b_per_w // C
    NBUF = 2
    n_passes = n_chunks // NBUF
    assert n_chunks % NBUF == 0 and n_passes >= 2, (n_chunks, NBUF)
    mesh = plsc.VectorSubcoreMesh(core_axis_name="c", subcore_axis_name="s")

    @functools.partial(
        pl.kernel,
        mesh=mesh,
        out_type=jax.ShapeDtypeStruct((B, D), jnp.float32),
        scratch_types=[
            pltpu.VMEM((b_per_w,), jnp.int32),
            pltpu.VMEM((_L,), jnp.int32),
        ]
        + [pltpu.VMEM((C, D), jnp.float32) for _ in range(NBUF)]
        + [pltpu.SemaphoreType.DMA for _ in range(2 * NBUF)],
    )
    def k(x_hbm, idx_hbm, shift_hbm, out_hbm, idx_v, shift_v, *scratch):
        bufs = scratch[:NBUF]
        gsems = scratch[NBUF:2 * NBUF]
        ssems = scratch[2 * NBUF:]
        wid = lax.axis_index("s") * _NC + lax.axis_index("c")
        base = wid * b_per_w
        pltpu.sync_copy(idx_hbm.at[pl.ds(base, b_per_w)], idx_v)
        pltpu.sync_copy(shift_hbm, shift_v)
        sh = shift_v[...]

        def adjust(g):
            for j in range(C // _L):
                sl = pl.ds(g * C + j * _L, _L)
                v = idx_v[sl] - sh
                idx_v[sl] = jnp.where(v < 0, v + N, v)

        def start_gather(g, b):
            pltpu.async_copy(
                x_hbm.at[idx_v.at[pl.ds(g * C, C)]], bufs[b], gsems[b]
            )

        def start_scatter(g, b):
            pltpu.async_copy(
                bufs[b], out_hbm.at[pl.ds(base + g * C, C)], ssems[b]
            )

        def wait_gather(g, b):
            pltpu.make_async_copy(
                x_hbm.at[idx_v.at[pl.ds(g * C, C)]], bufs[b], gsems[b]
            ).wait()

        def wait_scatter(g, b):
            pltpu.make_async_copy(
                bufs[b], out_hbm.at[pl.ds(base + g * C, C)], ssems[b]
            ).wait()

        for b in range(NBUF):
            adjust(b)
            start_gather(b, b)

        def body(p, carry):
            for b in range(NBUF):
                g = p * NBUF + b
                wait_gather(g, b)
                start_scatter(g, b)
                adjust(g + NBUF)
                wait_scatter(g, b)
                start_gather(g + NBUF, b)
            return carry

        lax.fori_loop(0, n_passes - 1, body, 0)

        g0 = (n_passes - 1) * NBUF
        for b in range(NBUF):
            wait_gather(g0 + b, b)
            start_scatter(g0 + b, b)
        for b in range(NBUF):
            wait_scatter(g0 + b, b)

    return k


def kernel(x, idx, shift):
    N, D = x.shape
    B = idx.shape[0]
    shift_vec = jnp.full(
        (_L,), jnp.asarray(shift, jnp.int32) % jnp.int32(N), dtype=jnp.int32
    )
    return _make_gather(N, D, B)(x, idx.astype(jnp.int32), shift_vec)


# speedup vs baseline: 1.0054x; 1.0054x over previous
import functools

import jax
import jax.numpy as jnp
from jax import lax
from jax.experimental import pallas as pl
from jax.experimental.pallas import tpu as pltpu
from jax.experimental.pallas import tpu_sc as plsc

_NC = 2
_NS = 16
_NW = _NC * _NS
_L = 16


@functools.lru_cache(maxsize=None)
def _make_gather(N, D, B):
    b_per_w = B // _NW
    C = 128
    n_chunks = b_per_w // C
    NBUF = 4
    PD = 2
    n_passes = (n_chunks - NBUF) // NBUF
    assert (n_chunks - NBUF) % NBUF == 0 and 0 < PD < NBUF and n_passes >= 1
    mesh = plsc.VectorSubcoreMesh(core_axis_name="c", subcore_axis_name="s")

    @functools.partial(
        pl.kernel,
        mesh=mesh,
        out_type=jax.ShapeDtypeStruct((B, D), jnp.float32),
        scratch_types=[
            pltpu.VMEM((b_per_w,), jnp.int32),
            pltpu.VMEM((_L,), jnp.int32),
        ]
        + [pltpu.VMEM((C, D), jnp.float32) for _ in range(NBUF)]
        + [pltpu.SemaphoreType.DMA for _ in range(2 * NBUF)],
    )
    def k(x_hbm, idx_hbm, shift_hbm, out_hbm, idx_v, shift_v, *scratch):
        bufs = scratch[:NBUF]
        gsems = scratch[NBUF:2 * NBUF]
        ssems = scratch[2 * NBUF:]
        wid = lax.axis_index("s") * _NC + lax.axis_index("c")
        base = wid * b_per_w
        pltpu.sync_copy(idx_hbm.at[pl.ds(base, b_per_w)], idx_v)
        pltpu.sync_copy(shift_hbm, shift_v)
        sh = shift_v[...]

        def adjust(g):
            for j in range(C // _L):
                sl = pl.ds(g * C + j * _L, _L)
                v = idx_v[sl] - sh
                idx_v[sl] = jnp.where(v < 0, v + N, v)

        def start_gather(g, b):
            pltpu.async_copy(
                x_hbm.at[idx_v.at[pl.ds(g * C, C)]], bufs[b], gsems[b]
            )

        def start_scatter(g, b):
            pltpu.async_copy(
                bufs[b], out_hbm.at[pl.ds(base + g * C, C)], ssems[b]
            )

        def wait_gather(g, b):
            pltpu.make_async_copy(
                x_hbm.at[idx_v.at[pl.ds(g * C, C)]], bufs[b], gsems[b]
            ).wait()

        def wait_scatter(g, b):
            pltpu.make_async_copy(
                bufs[b], out_hbm.at[pl.ds(base + g * C, C)], ssems[b]
            ).wait()

        def step(g, r, drain):
            bg = r
            bn = (r + PD) % NBUF
            wait_gather(g, bg)
            start_scatter(g, bg)
            adjust(g + PD)
            if drain:
                wait_scatter(g + PD - NBUF, bn)
            start_gather(g + PD, bn)

        for g in range(PD):
            adjust(g)
            start_gather(g, g % NBUF)
        for g in range(NBUF - PD):
            step(g, g % NBUF, drain=False)

        def body(p, carry):
            for b in range(NBUF):
                step((NBUF - PD) + p * NBUF + b,
                     ((NBUF - PD) + b) % NBUF, drain=True)
            return carry

        lax.fori_loop(0, n_passes, body, 0)

        for g in range(n_chunks - PD, n_chunks):
            wait_gather(g, g % NBUF)
            start_scatter(g, g % NBUF)
        for g in range(n_chunks - NBUF, n_chunks):
            wait_scatter(g, g % NBUF)

    return k


def kernel(x, idx, shift):
    N, D = x.shape
    B = idx.shape[0]
    shift_vec = jnp.full(
        (_L,), jnp.asarray(shift, jnp.int32) % jnp.int32(N), dtype=jnp.int32
    )
    return _make_gather(N, D, B)(x, idx.astype(jnp.int32), shift_vec)


# speedup vs baseline: 1.0104x; 1.0050x over previous
import functools

import jax
import jax.numpy as jnp
from jax import lax
from jax.experimental import pallas as pl
from jax.experimental.pallas import tpu as pltpu
from jax.experimental.pallas import tpu_sc as plsc

_NC = 2
_NS = 16
_NW = _NC * _NS
_L = 16


@functools.lru_cache(maxsize=None)
def _make_gather(N, D, B):
    b_per_w = B // _NW
    C = 64
    n_chunks = b_per_w // C
    NBUF = 8
    PD = 4
    n_passes = (n_chunks - NBUF) // NBUF
    assert (n_chunks - NBUF) % NBUF == 0 and 0 < PD < NBUF and n_passes >= 1
    mesh = plsc.VectorSubcoreMesh(core_axis_name="c", subcore_axis_name="s")

    @functools.partial(
        pl.kernel,
        mesh=mesh,
        out_type=jax.ShapeDtypeStruct((B, D), jnp.float32),
        scratch_types=[
            pltpu.VMEM((b_per_w,), jnp.int32),
            pltpu.VMEM((_L,), jnp.int32),
        ]
        + [pltpu.VMEM((C, D), jnp.float32) for _ in range(NBUF)]
        + [pltpu.SemaphoreType.DMA for _ in range(2 * NBUF)],
    )
    def k(x_hbm, idx_hbm, shift_hbm, out_hbm, idx_v, shift_v, *scratch):
        bufs = scratch[:NBUF]
        gsems = scratch[NBUF:2 * NBUF]
        ssems = scratch[2 * NBUF:]
        wid = lax.axis_index("s") * _NC + lax.axis_index("c")
        base = wid * b_per_w
        pltpu.sync_copy(idx_hbm.at[pl.ds(base, b_per_w)], idx_v)
        pltpu.sync_copy(shift_hbm, shift_v)
        sh = shift_v[...]

        def adjust(g):
            for j in range(C // _L):
                sl = pl.ds(g * C + j * _L, _L)
                v = idx_v[sl] - sh
                idx_v[sl] = jnp.where(v < 0, v + N, v)

        def start_gather(g, b):
            pltpu.async_copy(
                x_hbm.at[idx_v.at[pl.ds(g * C, C)]], bufs[b], gsems[b]
            )

        def start_scatter(g, b):
            pltpu.async_copy(
                bufs[b], out_hbm.at[pl.ds(base + g * C, C)], ssems[b]
            )

        def wait_gather(g, b):
            pltpu.make_async_copy(
                x_hbm.at[idx_v.at[pl.ds(g * C, C)]], bufs[b], gsems[b]
            ).wait()

        def wait_scatter(g, b):
            pltpu.make_async_copy(
                bufs[b], out_hbm.at[pl.ds(base + g * C, C)], ssems[b]
            ).wait()

        def step(g, r, drain):
            bg = r
            bn = (r + PD) % NBUF
            wait_gather(g, bg)
            start_scatter(g, bg)
            adjust(g + PD)
            if drain:
                wait_scatter(g + PD - NBUF, bn)
            start_gather(g + PD, bn)

        for g in range(PD):
            adjust(g)
            start_gather(g, g % NBUF)
        for g in range(NBUF - PD):
            step(g, g % NBUF, drain=False)

        def body(p, carry):
            for b in range(NBUF):
                step((NBUF - PD) + p * NBUF + b,
                     ((NBUF - PD) + b) % NBUF, drain=True)
            return carry

        lax.fori_loop(0, n_passes, body, 0)

        for g in range(n_chunks - PD, n_chunks):
            wait_gather(g, g % NBUF)
            start_scatter(g, g % NBUF)
        for g in range(n_chunks - NBUF, n_chunks):
            wait_scatter(g, g % NBUF)

    return k


def kernel(x, idx, shift):
    N, D = x.shape
    B = idx.shape[0]
    shift_vec = jnp.full(
        (_L,), jnp.asarray(shift, jnp.int32) % jnp.int32(N), dtype=jnp.int32
    )
    return _make_gather(N, D, B)(x, idx.astype(jnp.int32), shift_vec)
